# baseline (speedup 1.0000x reference)
import jax
import jax.numpy as jnp
from jax.experimental import pallas as pl

_NODE_IN, _EDGE_IN, _HID = 128, 16, 500
_HP = 512
_B_RXN, _C, _N_PER, _E_PER = 50, 20, 50, 100
_V1, _E1 = _B_RXN * _N_PER, _B_RXN * _E_PER
_B = _B_RXN * _C
_V2, _E2 = _B * _N_PER, _B * _E_PER
_N_LAYERS = 3
_NG = _B_RXN + _B

_G_ENC = 30
_S_SUB = 2
_R_DIFF = 2

_F32 = jnp.float32
_BF16 = jnp.bfloat16


def _pad2(w, rows, cols):
    return jnp.pad(w, ((0, rows - w.shape[0]), (0, cols - w.shape[1])))


def _pad_bias(b):
    return jnp.pad(b, (0, _HP - b.shape[0])).reshape(1, _HP)


def _onehots(src_row, dst_row, n_blk, n_sub, e_sub):
    iota_n = jax.lax.broadcasted_iota(jnp.int32, (n_sub, e_sub), 0)
    ohg, ohd = [], []
    for s in range(n_blk // n_sub):
        src_s = src_row[:, s * e_sub:(s + 1) * e_sub] - (s * n_sub)
        dst_s = dst_row[:, s * e_sub:(s + 1) * e_sub] - (s * n_sub)
        ohg.append((jnp.broadcast_to(src_s, (n_sub, e_sub)) == iota_n)
                   .astype(_F32))
        ohd.append((jnp.broadcast_to(dst_s, (n_sub, e_sub)) == iota_n)
                   .astype(_F32))
    return ohg, ohd


def _mp_layer(h, ew, ohg, ohd, mW1, nW1, nW2, nb, n_sub, e_sub):
    n_blk = h.shape[0]
    hw = jnp.dot(h, mW1, preferred_element_type=_F32)
    msums = []
    for s in range(n_blk // n_sub):
        hw_s = hw[s * n_sub:(s + 1) * n_sub, :]
        hsrc = jax.lax.dot_general(ohg[s], hw_s, (((0,), (0,)), ((), ())),
                                   preferred_element_type=_F32)
        msg = jnp.maximum(hsrc + ew[s * e_sub:(s + 1) * e_sub, :], 0.0)
        msums.append(jnp.dot(ohd[s], msg, preferred_element_type=_F32))
    m_sum = jnp.concatenate(msums, axis=0) if len(msums) > 1 else msums[0]
    return jnp.maximum(jnp.dot(h, nW1, preferred_element_type=_F32)
                       + jnp.dot(m_sum, nW2, preferred_element_type=_F32)
                       + nb, 0.0)


def _enc_body(nf, e, srcr, dstr, *args):
    out = args[-1]
    pW, pb = args[0], args[1]
    lw = args[2:-1]
    n_blk = _G_ENC * _N_PER
    base = pl.program_id(0) * n_blk
    src_row = srcr[0] - base
    dst_row = dstr[0] - base
    h = jnp.maximum(jnp.dot(nf[0], pW[...], preferred_element_type=_F32)
                    + pb[...], 0.0)
    eb = e[0]
    n_sub, e_sub = _S_SUB * _N_PER, _S_SUB * _E_PER
    ohg, ohd = _onehots(src_row, dst_row, n_blk, n_sub, e_sub)
    for i in range(_N_LAYERS):
        mW1, mW2, mb, nW1, nW2, nb = lw[6 * i:6 * i + 6]
        ew = jnp.dot(eb, mW2[...], preferred_element_type=_F32) + mb[...]
        h = _mp_layer(h, ew, ohg, ohd, mW1[...], nW1[...], nW2[...],
                      nb[...], n_sub, e_sub)
    out[0] = h


def _diff_body(r_h, p_h, e, srcr, dstr, dmW1, dmW2, dmb, dnW1, dnW2, dnb,
               pW1, pb1, pW2, pb2, cs, out):
    n_blk = _R_DIFF * _C * _N_PER
    n_r = _R_DIFF * _N_PER
    n_i = jax.lax.broadcasted_iota(jnp.int32, (n_blk, n_r), 0)
    i_i = jax.lax.broadcasted_iota(jnp.int32, (n_blk, n_r), 1)
    tgt = (n_i // (_C * _N_PER)) * _N_PER + (n_i % _N_PER)
    rep_oh = (tgt == i_i).astype(_F32)
    diff = p_h[0] - jnp.dot(rep_oh, r_h[0], preferred_element_type=_F32)
    base = pl.program_id(0) * n_blk
    src_row = srcr[0] - base
    dst_row = dstr[0] - base
    ew = jnp.dot(e[0], dmW2[...], preferred_element_type=_F32) + dmb[...]
    n_sub, e_sub = _S_SUB * _N_PER, _S_SUB * _E_PER
    ohg, ohd = _onehots(src_row, dst_row, n_blk, n_sub, e_sub)
    h = _mp_layer(diff, ew, ohg, ohd, dmW1[...], dnW1[...], dnW2[...],
                  dnb[...], n_sub, e_sub)
    ng = _R_DIFF * _C
    g_i = jax.lax.broadcasted_iota(jnp.int32, (ng, n_blk), 0)
    n_i2 = jax.lax.broadcasted_iota(jnp.int32, (ng, n_blk), 1)
    sum_oh = (n_i2 // _N_PER == g_i).astype(_F32)
    readout = jnp.dot(sum_oh, h, preferred_element_type=_F32)
    hidden = jnp.maximum(jnp.dot(readout, pW1[...], preferred_element_type=_F32)
                         + pb1[...], 0.0)
    out[0] = (jnp.dot(hidden, pW2[...], preferred_element_type=_F32)
              + pb2[...] + cs[0])


def _full_spec(shape):
    nd = len(shape)
    return pl.BlockSpec(shape, lambda i: (0,) * nd)


def kernel(reactant_node_feats, reactant_edge_feats, product_node_feats,
           product_edge_feats, candidate_scores, reactant_edge_index,
           product_edge_index, params):
    p = params
    nf = jnp.concatenate([reactant_node_feats, product_node_feats], axis=0)
    ef = jnp.concatenate([reactant_edge_feats, product_edge_feats], axis=0)
    src = jnp.concatenate([reactant_edge_index[0], product_edge_index[0] + _V1])
    dst = jnp.concatenate([reactant_edge_index[1], product_edge_index[1] + _V1])

    nbe = _NG // _G_ENC
    n_blk, e_blk = _G_ENC * _N_PER, _G_ENC * _E_PER
    nf3 = nf.reshape(nbe, n_blk, _NODE_IN)
    ef3 = ef.reshape(nbe, e_blk, _EDGE_IN)
    src3 = src.reshape(nbe, 1, e_blk)
    dst3 = dst.reshape(nbe, 1, e_blk)

    pW = _pad2(p['proj_W'], _NODE_IN, _HP)
    pb = _pad_bias(p['proj_b'])
    enc_w = [pW, pb]
    for i in range(_N_LAYERS):
        mW = p['msg_W_%d' % i]
        nW = p['node_W_%d' % i]
        enc_w += [_pad2(mW[:_HID], _HP, _HP),
                  _pad2(mW[_HID:], _EDGE_IN, _HP),
                  _pad_bias(p['msg_b_%d' % i]),
                  _pad2(nW[:_HID], _HP, _HP),
                  _pad2(nW[_HID:], _HP, _HP),
                  _pad_bias(p['node_b_%d' % i])]

    enc_specs = (
        [pl.BlockSpec((1, n_blk, _NODE_IN), lambda i: (i, 0, 0)),
         pl.BlockSpec((1, e_blk, _EDGE_IN), lambda i: (i, 0, 0)),
         pl.BlockSpec((1, 1, e_blk), lambda i: (i, 0, 0)),
         pl.BlockSpec((1, 1, e_blk), lambda i: (i, 0, 0))]
        + [_full_spec(w.shape) for w in enc_w])

    h_all = pl.pallas_call(
        _enc_body,
        grid=(nbe,),
        in_specs=enc_specs,
        out_specs=pl.BlockSpec((1, n_blk, _HP), lambda i: (i, 0, 0)),
        out_shape=jax.ShapeDtypeStruct((nbe, n_blk, _HP), _F32),
    )(nf3, ef3, src3, dst3, *enc_w)
    h_all = h_all.reshape(_V1 + _V2, _HP)

    nbd = _B_RXN // _R_DIFF
    nd_blk = _R_DIFF * _C * _N_PER
    ed_blk = _R_DIFF * _C * _E_PER
    r_h3 = h_all[:_V1].reshape(nbd, _R_DIFF * _N_PER, _HP)
    p_h3 = h_all[_V1:].reshape(nbd, nd_blk, _HP)
    pe3 = product_edge_feats.reshape(nbd, ed_blk, _EDGE_IN)
    ps3 = product_edge_index[0].reshape(nbd, 1, ed_blk)
    pd3 = product_edge_index[1].reshape(nbd, 1, ed_blk)
    cs3 = jnp.pad(candidate_scores, ((0, 0), (0, 127))).reshape(
        nbd, _R_DIFF * _C, 128)

    dmW = p['dmsg_W']
    dnW = p['dnode_W']
    diff_w = [_pad2(dmW[:_HID], _HP, _HP),
              _pad2(dmW[_HID:], _EDGE_IN, _HP),
              _pad_bias(p['dmsg_b']),
              _pad2(dnW[:_HID], _HP, _HP),
              _pad2(dnW[_HID:], _HP, _HP),
              _pad_bias(p['dnode_b']),
              _pad2(p['pW1'], _HP, _HP), _pad_bias(p['pb1']),
              _pad2(p['pW2'], _HP, 128),
              jnp.pad(p['pb2'], (0, 127)).reshape(1, 128)]

    diff_specs = (
        [pl.BlockSpec((1, _R_DIFF * _N_PER, _HP), lambda i: (i, 0, 0)),
         pl.BlockSpec((1, nd_blk, _HP), lambda i: (i, 0, 0)),
         pl.BlockSpec((1, ed_blk, _EDGE_IN), lambda i: (i, 0, 0)),
         pl.BlockSpec((1, 1, ed_blk), lambda i: (i, 0, 0)),
         pl.BlockSpec((1, 1, ed_blk), lambda i: (i, 0, 0))]
        + [_full_spec(w.shape) for w in diff_w]
        + [pl.BlockSpec((1, _R_DIFF * _C, 128), lambda i: (i, 0, 0))])

    scores = pl.pallas_call(
        _diff_body,
        grid=(nbd,),
        in_specs=diff_specs,
        out_specs=pl.BlockSpec((1, _R_DIFF * _C, 128), lambda i: (i, 0, 0)),
        out_shape=jax.ShapeDtypeStruct((nbd, _R_DIFF * _C, 128), _F32),
    )(r_h3, p_h3, pe3, ps3, pd3, *diff_w, cs3)

    return scores.reshape(_B, 128)[:, :1]


# speedup vs baseline: 1.1863x; 1.1863x over previous
import jax
import jax.numpy as jnp
from jax.experimental import pallas as pl

_NODE_IN, _EDGE_IN, _HID = 128, 16, 500
_HP = 512
_B_RXN, _C, _N_PER, _E_PER = 50, 20, 50, 100
_V1, _E1 = _B_RXN * _N_PER, _B_RXN * _E_PER
_B = _B_RXN * _C
_V2, _E2 = _B * _N_PER, _B * _E_PER
_N_LAYERS = 3

_NB = (1 + _C) * _N_PER
_EB = (1 + _C) * _E_PER
_S_ENC = 3
_S_D = 2

_F32 = jnp.float32


def _pad2(w, rows, cols):
    return jnp.pad(w, ((0, rows - w.shape[0]), (0, cols - w.shape[1])))


def _pad_bias(b):
    return jnp.pad(b, (0, _HP - b.shape[0])).reshape(1, _HP)


def _onehots(src_row, dst_row, n_blk, n_sub, e_sub):
    iota_n = jax.lax.broadcasted_iota(jnp.int32, (n_sub, e_sub), 0)
    ohg, ohd = [], []
    for s in range(n_blk // n_sub):
        src_s = src_row[:, s * e_sub:(s + 1) * e_sub] - (s * n_sub)
        dst_s = dst_row[:, s * e_sub:(s + 1) * e_sub] - (s * n_sub)
        ohg.append((jnp.broadcast_to(src_s, (n_sub, e_sub)) == iota_n)
                   .astype(_F32))
        ohd.append((jnp.broadcast_to(dst_s, (n_sub, e_sub)) == iota_n)
                   .astype(_F32))
    return ohg, ohd


def _mp_layer(h, ew, ohg, ohd, mW1, nW1, nW2, nb, n_sub, e_sub):
    n_blk = h.shape[0]
    hw = jnp.dot(h, mW1, preferred_element_type=_F32)
    msums = []
    for s in range(n_blk // n_sub):
        hw_s = hw[s * n_sub:(s + 1) * n_sub, :]
        hsrc = jax.lax.dot_general(ohg[s], hw_s, (((0,), (0,)), ((), ())),
                                   preferred_element_type=_F32)
        msg = jnp.maximum(hsrc + ew[s * e_sub:(s + 1) * e_sub, :], 0.0)
        msums.append(jnp.dot(ohd[s], msg, preferred_element_type=_F32))
    m_sum = jnp.concatenate(msums, axis=0) if len(msums) > 1 else msums[0]
    return jnp.maximum(jnp.dot(h, nW1, preferred_element_type=_F32)
                       + jnp.dot(m_sum, nW2, preferred_element_type=_F32)
                       + nb, 0.0)


def _body(nf, ef, srcr, dstr, cs, *args):
    out = args[-1]
    pW, pb = args[0], args[1]
    lw = args[2:2 + 6 * _N_LAYERS]
    dmW1, dmW2, dmb, dnW1, dnW2, dnb, pW1, pb1, pW2, pb2 = \
        args[2 + 6 * _N_LAYERS:-1]
    src_row = srcr[0]
    dst_row = dstr[0]
    h = jnp.maximum(jnp.dot(nf[0], pW[...], preferred_element_type=_F32)
                    + pb[...], 0.0)
    eb = ef[0]
    n_sub, e_sub = _S_ENC * _N_PER, _S_ENC * _E_PER
    ohg, ohd = _onehots(src_row, dst_row, _NB, n_sub, e_sub)
    for i in range(_N_LAYERS):
        mW1, mW2, mb, nW1, nW2, nb = lw[6 * i:6 * i + 6]
        ew = jnp.dot(eb, mW2[...], preferred_element_type=_F32) + mb[...]
        h = _mp_layer(h, ew, ohg, ohd, mW1[...], nW1[...], nW2[...],
                      nb[...], n_sub, e_sub)
    n_p = _C * _N_PER
    n_i = jax.lax.broadcasted_iota(jnp.int32, (n_p, _N_PER), 0)
    i_i = jax.lax.broadcasted_iota(jnp.int32, (n_p, _N_PER), 1)
    rep_oh = (n_i % _N_PER == i_i).astype(_F32)
    diff = h[_N_PER:, :] - jnp.dot(rep_oh, h[:_N_PER, :],
                                   preferred_element_type=_F32)
    ps_row = src_row[:, _E_PER:] - _N_PER
    pd_row = dst_row[:, _E_PER:] - _N_PER
    n_sub_d, e_sub_d = _S_D * _N_PER, _S_D * _E_PER
    ohg_d, ohd_d = _onehots(ps_row, pd_row, n_p, n_sub_d, e_sub_d)
    ew_d = (jnp.dot(eb[_E_PER:, :], dmW2[...], preferred_element_type=_F32)
            + dmb[...])
    h2 = _mp_layer(diff, ew_d, ohg_d, ohd_d, dmW1[...], dnW1[...], dnW2[...],
                   dnb[...], n_sub_d, e_sub_d)
    g_i = jax.lax.broadcasted_iota(jnp.int32, (_C, n_p), 0)
    n_i2 = jax.lax.broadcasted_iota(jnp.int32, (_C, n_p), 1)
    sum_oh = (n_i2 // _N_PER == g_i).astype(_F32)
    readout = jnp.dot(sum_oh, h2, preferred_element_type=_F32)
    hidden = jnp.maximum(jnp.dot(readout, pW1[...], preferred_element_type=_F32)
                         + pb1[...], 0.0)
    out[0] = (jnp.dot(hidden, pW2[...], preferred_element_type=_F32)
              + pb2[...] + cs[0])


def _full_spec(shape):
    nd = len(shape)
    return pl.BlockSpec(shape, lambda i: (0,) * nd)


def kernel(reactant_node_feats, reactant_edge_feats, product_node_feats,
           product_edge_feats, candidate_scores, reactant_edge_index,
           product_edge_index, params):
    p = params
    nf3 = jnp.concatenate(
        [reactant_node_feats.reshape(_B_RXN, _N_PER, _NODE_IN),
         product_node_feats.reshape(_B_RXN, _C * _N_PER, _NODE_IN)], axis=1)
    ef3 = jnp.concatenate(
        [reactant_edge_feats.reshape(_B_RXN, _E_PER, _EDGE_IN),
         product_edge_feats.reshape(_B_RXN, _C * _E_PER, _EDGE_IN)], axis=1)
    r_off = (jnp.arange(_B_RXN, dtype=jnp.int32) * _N_PER)[:, None]
    p_off = (jnp.arange(_B_RXN, dtype=jnp.int32) * (_C * _N_PER))[:, None]
    src3 = jnp.concatenate(
        [reactant_edge_index[0].reshape(_B_RXN, _E_PER) - r_off,
         product_edge_index[0].reshape(_B_RXN, _C * _E_PER) - p_off + _N_PER],
        axis=1).reshape(_B_RXN, 1, _EB)
    dst3 = jnp.concatenate(
        [reactant_edge_index[1].reshape(_B_RXN, _E_PER) - r_off,
         product_edge_index[1].reshape(_B_RXN, _C * _E_PER) - p_off + _N_PER],
        axis=1).reshape(_B_RXN, 1, _EB)
    cs3 = jnp.pad(candidate_scores, ((0, 0), (0, 127))).reshape(_B_RXN, _C, 128)

    pW = _pad2(p['proj_W'], _NODE_IN, _HP)
    pb = _pad_bias(p['proj_b'])
    ws = [pW, pb]
    for i in range(_N_LAYERS):
        mW = p['msg_W_%d' % i]
        nW = p['node_W_%d' % i]
        ws += [_pad2(mW[:_HID], _HP, _HP), _pad2(mW[_HID:], _EDGE_IN, _HP),
               _pad_bias(p['msg_b_%d' % i]), _pad2(nW[:_HID], _HP, _HP),
               _pad2(nW[_HID:], _HP, _HP), _pad_bias(p['node_b_%d' % i])]
    dmW = p['dmsg_W']
    dnW = p['dnode_W']
    ws += [_pad2(dmW[:_HID], _HP, _HP), _pad2(dmW[_HID:], _EDGE_IN, _HP),
           _pad_bias(p['dmsg_b']), _pad2(dnW[:_HID], _HP, _HP),
           _pad2(dnW[_HID:], _HP, _HP), _pad_bias(p['dnode_b']),
           _pad2(p['pW1'], _HP, _HP), _pad_bias(p['pb1']),
           _pad2(p['pW2'], _HP, 128),
           jnp.pad(p['pb2'], (0, 127)).reshape(1, 128)]

    specs = (
        [pl.BlockSpec((1, _NB, _NODE_IN), lambda i: (i, 0, 0)),
         pl.BlockSpec((1, _EB, _EDGE_IN), lambda i: (i, 0, 0)),
         pl.BlockSpec((1, 1, _EB), lambda i: (i, 0, 0)),
         pl.BlockSpec((1, 1, _EB), lambda i: (i, 0, 0)),
         pl.BlockSpec((1, _C, 128), lambda i: (i, 0, 0))]
        + [_full_spec(w.shape) for w in ws])

    scores = pl.pallas_call(
        _body,
        grid=(_B_RXN,),
        in_specs=specs,
        out_specs=pl.BlockSpec((1, _C, 128), lambda i: (i, 0, 0)),
        out_shape=jax.ShapeDtypeStruct((_B_RXN, _C, 128), _F32),
    )(nf3, ef3, src3, dst3, cs3, *ws)

    return scores.reshape(_B, 128)[:, :1]
